# baseline (speedup 1.0000x reference)
import functools

import jax
import jax.numpy as jnp
from jax import lax
from jax.experimental import pallas as pl
from jax.experimental.pallas import tpu as pltpu
from jax.experimental.pallas import tpu_sc as plsc

_N = 20000
_NPAD = 20480
_C = 21
_CPAD = 24
_B = 8
_ROWS = _B * _CPAD
_PRE = 600
_PREPAD = 640
_SORTW = 1024
_INST = _B * _C
_LANES = 256
_MAXT = 200
_SCORE_TH = 0.5
_IOU_TH = 0.5
_BASE = 0x3F000000
_G = 16


def _decode_body(deltas_ref, priors_ref, probs_ref, boxes_ref, scores_ref, t_ref, i_ref, dest_ref):
    d = deltas_ref[0]
    p = priors_ref[...]
    pw = p[3:4] - p[1:2]
    ph = p[2:3] - p[0:1]
    pcx = p[1:2] + 0.5 * pw
    pcy = p[0:1] + 0.5 * ph
    bw = jnp.exp(d[3:4] * 0.2) * pw
    bh = jnp.exp(d[2:3] * 0.2) * ph
    bcx = (d[1:2] * 0.1) * pw + pcx
    bcy = (d[0:1] * 0.1) * ph + pcy
    y1 = bcy - 0.5 * bh
    x1 = bcx - 0.5 * bw
    y2 = bh + y1
    x2 = bw + x1
    boxes_ref[0] = jnp.clip(jnp.concatenate([y1, x1, y2, x2], axis=0), 0.0, 1.0)
    pr = probs_ref[0]
    mx = jnp.max(pr, axis=0, keepdims=True)
    nonbg = pr[0:1] < mx
    sc = jnp.where(nonbg, pr, 0.0)
    scores_ref[0] = sc

    keys = lax.bitcast_convert_type(sc, jnp.int32)
    t = jnp.full((_CPAD, 1), _BASE, jnp.int32)
    for bit in range(22, -1, -1):
        cand = t | (1 << bit)
        cnt = jnp.sum((keys > cand).astype(jnp.int32), axis=1, keepdims=True)
        t = jnp.where(cnt >= _PRE, cand, t)
    big_t = t + 1
    m = jnp.sum((keys > big_t).astype(jnp.int32), axis=1, keepdims=True)
    need = _PRE - m
    tie = keys == big_t
    iota = lax.broadcasted_iota(jnp.int32, (_CPAD, _NPAD), 1)
    sel_i = jnp.zeros((_CPAD, 1), jnp.int32)
    for bit in range(14, -1, -1):
        cand_i = sel_i + (1 << bit)
        cnt = jnp.sum((tie & (iota < cand_i)).astype(jnp.int32), axis=1, keepdims=True)
        sel_i = jnp.where(cnt < need, cand_i, sel_i)
    b = pl.program_id(0)
    mask = (keys > big_t) | (tie & (iota <= sel_i))
    m = mask.astype(jnp.int32)
    s = 1
    while s < _NPAD:
        m = m + jnp.where(iota >= s, pltpu.roll(m, s, 1), 0)
        s *= 2
    rowbase = (b * _CPAD + lax.broadcasted_iota(jnp.int32, (_CPAD, _NPAD), 0)) * _PREPAD
    dest = jnp.where(mask, jnp.minimum(m - 1, _PREPAD - 1), (_PRE + 8) + (iota & 15))
    dest_ref[0] = dest + rowbase
    t_ref[0] = jnp.broadcast_to(big_t, (_CPAD, 128))
    i_ref[0] = jnp.broadcast_to(sel_i, (_CPAD, 128))


def _decode(pred_deltas, pred_label_probs, prior_boxes):
    deltas_t = jnp.pad(jnp.swapaxes(pred_deltas, 1, 2), ((0, 0), (0, 0), (0, _NPAD - _N)))
    priors_t = jnp.pad(prior_boxes.T, ((0, 0), (0, _NPAD - _N)))
    probs_t = jnp.pad(
        jnp.swapaxes(pred_label_probs, 1, 2),
        ((0, 0), (0, _CPAD - _C), (0, _NPAD - _N)),
        constant_values=-1.0,
    )
    return pl.pallas_call(
        _decode_body,
        grid=(_B,),
        in_specs=[
            pl.BlockSpec((1, 4, _NPAD), lambda b: (b, 0, 0)),
            pl.BlockSpec((4, _NPAD), lambda b: (0, 0)),
            pl.BlockSpec((1, _CPAD, _NPAD), lambda b: (b, 0, 0)),
        ],
        out_specs=[
            pl.BlockSpec((1, 4, _NPAD), lambda b: (b, 0, 0)),
            pl.BlockSpec((1, _CPAD, _NPAD), lambda b: (b, 0, 0)),
            pl.BlockSpec((1, _CPAD, 128), lambda b: (b, 0, 0)),
            pl.BlockSpec((1, _CPAD, 128), lambda b: (b, 0, 0)),
            pl.BlockSpec((1, _CPAD, _NPAD), lambda b: (b, 0, 0)),
        ],
        out_shape=[
            jax.ShapeDtypeStruct((_B, 4, _NPAD), jnp.float32),
            jax.ShapeDtypeStruct((_B, _CPAD, _NPAD), jnp.float32),
            jax.ShapeDtypeStruct((_B, _CPAD, 128), jnp.int32),
            jax.ShapeDtypeStruct((_B, _CPAD, 128), jnp.int32),
            jax.ShapeDtypeStruct((_B, _CPAD, _NPAD), jnp.int32),
        ],
    )(deltas_t, priors_t, probs_t)


def _make_sc_compact():
    info = plsc.get_sparse_core_info()
    nw = info.num_cores * info.num_subcores
    units = _ROWS // nw
    mesh = plsc.VectorSubcoreMesh(core_axis_name="c", subcore_axis_name="s")

    @functools.partial(
        pl.kernel,
        mesh=mesh,
        out_type=[
            jax.ShapeDtypeStruct((_ROWS * _PREPAD,), jnp.int32),
            jax.ShapeDtypeStruct((_ROWS * _PREPAD,), jnp.int32),
        ],
        scratch_types=[
            pltpu.VMEM((_NPAD,), jnp.int32),
            pltpu.VMEM((_NPAD,), jnp.int32),
            pltpu.VMEM((_NPAD,), jnp.int32),
            pltpu.VMEM((_PREPAD,), jnp.int32),
            pltpu.VMEM((_PREPAD,), jnp.int32),
            pltpu.VMEM((128,), jnp.int32),
            pltpu.VMEM((128,), jnp.int32),
            pltpu.SemaphoreType.DMA,
            pltpu.SemaphoreType.DMA,
        ],
    )
    def sc_compact(
        keys_hbm, dest_hbm, iota_hbm, okey_hbm, oidx_hbm,
        row_v, drow_v, iot_v, neg1_v, zero_v, didx_a, didx_b, sem_a, sem_b,
    ):
        wid = lax.axis_index("s") * info.num_cores + lax.axis_index("c")
        pltpu.sync_copy(iota_hbm, iot_v)

        def fillconst(c, _):
            neg1_v[pl.ds(c * 16, 16)] = jnp.full((16,), -1, jnp.int32)
            zero_v[pl.ds(c * 16, 16)] = jnp.zeros((16,), jnp.int32)
            return 0

        lax.fori_loop(0, _PREPAD // 16, fillconst, 0)

        for u in range(units):
            r = wid * units + u
            base = r * _PREPAD
            pltpu.sync_copy(neg1_v, okey_hbm.at[pl.ds(base, _PREPAD)])
            pltpu.sync_copy(zero_v, oidx_hbm.at[pl.ds(base, _PREPAD)])
            pltpu.sync_copy(keys_hbm.at[r], row_v)
            pltpu.sync_copy(dest_hbm.at[r], drow_v)

            def grp(g, _):
                ga = g * 256
                gb = ga + 128
                for cc in range(8):
                    didx_a[pl.ds(cc * 16, 16)] = drow_v[pl.ds(ga + cc * 16, 16)]
                ca1 = pltpu.async_copy(row_v.at[pl.ds(ga, 128)], okey_hbm.at[didx_a], sem_a)
                ca2 = pltpu.async_copy(iot_v.at[pl.ds(ga, 128)], oidx_hbm.at[didx_a], sem_a)
                for cc in range(8):
                    didx_b[pl.ds(cc * 16, 16)] = drow_v[pl.ds(gb + cc * 16, 16)]
                cb1 = pltpu.async_copy(row_v.at[pl.ds(gb, 128)], okey_hbm.at[didx_b], sem_b)
                cb2 = pltpu.async_copy(iot_v.at[pl.ds(gb, 128)], oidx_hbm.at[didx_b], sem_b)
                ca1.wait()
                ca2.wait()
                cb1.wait()
                cb2.wait()
                return 0

            lax.fori_loop(0, _NPAD // 256, grp, 0)

    return sc_compact


def _sort_body(key_ref, idx_ref, sc_out_ref, idx_out_ref):
    key = key_ref[...]
    idx = idx_ref[...]
    pos = lax.broadcasted_iota(jnp.int32, (_ROWS, _SORTW), 1)
    k = 2
    while k <= _SORTW:
        desc = (pos & k) == 0
        j = k // 2
        while j >= 1:
            first = (pos & j) == 0
            keep_better = first == desc
            pk = jnp.where(first, pltpu.roll(key, _SORTW - j, 1), pltpu.roll(key, j, 1))
            pi = jnp.where(first, pltpu.roll(idx, _SORTW - j, 1), pltpu.roll(idx, j, 1))
            g = (key > pk) | ((key == pk) & (idx < pi))
            selcur = g == keep_better
            key = jnp.where(selcur, key, pk)
            idx = jnp.where(selcur, idx, pi)
            j //= 2
        k *= 2
    sc_out_ref[...] = jnp.where(key < 0, -1.0, lax.bitcast_convert_type(key, jnp.float32))
    idx_out_ref[...] = idx


def _sort(okey, oidx):
    return pl.pallas_call(
        _sort_body,
        out_shape=[
            jax.ShapeDtypeStruct((_ROWS, _SORTW), jnp.float32),
            jax.ShapeDtypeStruct((_ROWS, _SORTW), jnp.int32),
        ],
    )(okey, oidx)


def _nms_body(y1_ref, x1_ref, y2_ref, x2_ref, sc_ref, out_ref, area_ref, keep_ref):
    y1 = y1_ref[...]
    x1 = x1_ref[...]
    y2 = y2_ref[...]
    x2 = x2_ref[...]
    area_ref[...] = (y2 - y1) * (x2 - x1)
    keep_ref[...] = jnp.ones_like(y1)

    _BK = 64
    for b0 in range(0, _PRE, _BK):
        nsteps = min(_BK, _PRE - b0)
        suf = _PREPAD - b0
        sy1 = y1_ref[b0:, :]
        sx1 = x1_ref[b0:, :]
        sy2 = y2_ref[b0:, :]
        sx2 = x2_ref[b0:, :]
        sarea = area_ref[b0:, :]
        rows = lax.broadcasted_iota(jnp.int32, (suf, _LANES), 0) + b0

        def body(i, _, sy1=sy1, sx1=sx1, sy2=sy2, sx2=sx2, sarea=sarea, rows=rows, b0=b0):
            ry1 = y1_ref[pl.ds(i, 1), :]
            rx1 = x1_ref[pl.ds(i, 1), :]
            ry2 = y2_ref[pl.ds(i, 1), :]
            rx2 = x2_ref[pl.ds(i, 1), :]
            rsc = sc_ref[pl.ds(i, 1), :]
            rkeep = keep_ref[pl.ds(i, 1), :]
            rarea = area_ref[pl.ds(i, 1), :]
            can = (rkeep > 0.0) & (rsc > _SCORE_TH)
            ih = jnp.maximum(jnp.minimum(sy2, ry2) - jnp.maximum(sy1, ry1), 0.0)
            iw = jnp.maximum(jnp.minimum(sx2, rx2) - jnp.maximum(sx1, rx1), 0.0)
            inter = ih * iw
            union = sarea + rarea - inter
            iou = inter / jnp.maximum(union, 1e-8)
            sup = (iou > _IOU_TH) & (rows != i) & can
            keep_ref[b0:, :] = jnp.where(sup, 0.0, keep_ref[b0:, :])
            return 0

        lax.fori_loop(b0, b0 + nsteps, body, 0)
    sc = sc_ref[...]
    out_ref[...] = jnp.where((keep_ref[...] > 0.0) & (sc > _SCORE_TH), sc, -1.0)


def _nms(y1l, x1l, y2l, x2l, scl):
    return pl.pallas_call(
        _nms_body,
        out_shape=jax.ShapeDtypeStruct((_PREPAD, _LANES), jnp.float32),
        scratch_shapes=[
            pltpu.VMEM((_PREPAD, _LANES), jnp.float32),
            pltpu.VMEM((_PREPAD, _LANES), jnp.float32),
        ],
    )(y1l, x1l, y2l, x2l, scl)


def _to_lane(a, pad_val):
    a = a.reshape(_INST, _PREPAD).T
    return jnp.pad(a, ((0, 0), (0, _LANES - _INST)), constant_values=pad_val)


def kernel(pred_deltas, pred_label_probs, prior_boxes):
    boxes_t, scores_t, t_out, i_out, dest = _decode(pred_deltas, pred_label_probs, prior_boxes)
    keys = lax.bitcast_convert_type(scores_t, jnp.int32).reshape(_ROWS, _NPAD)
    dest_r = dest.reshape(_ROWS, _NPAD)
    iota_r = jnp.arange(_NPAD, dtype=jnp.int32)
    okey_f, oidx_f = _make_sc_compact()(keys, dest_r, iota_r)
    colok = jnp.arange(_PREPAD, dtype=jnp.int32)[None, :] < _PRE
    okey = jnp.where(colok, okey_f.reshape(_ROWS, _PREPAD), -1)
    oidx = jnp.where(colok, oidx_f.reshape(_ROWS, _PREPAD), 0)
    okey_p = jnp.pad(okey, ((0, 0), (0, _SORTW - _PREPAD)), constant_values=-1)
    oidx_p = jnp.pad(oidx, ((0, 0), (0, _SORTW - _PREPAD)), constant_values=0)
    ssc, sidx = _sort(okey_p, oidx_p)
    ssc = ssc[:, :_PREPAD].reshape(_B, _CPAD, _PREPAD)[:, :_C]
    sidx = sidx[:, :_PREPAD].reshape(_B, _CPAD, _PREPAD)[:, :_C]
    bsel = jnp.take_along_axis(boxes_t[:, :, None, :], sidx[:, None, :, :], axis=3)
    comps = [bsel[:, k] for k in range(4)]
    y1l, x1l, y2l, x2l = [_to_lane(c, 0.0) for c in comps]
    scl = _to_lane(ssc, -1.0)
    kept = _nms(y1l, x1l, y2l, x2l, scl)
    kept640 = kept[:, :_INST].T.reshape(_B, _C, _PREPAD)
    sel_scores, sidx2 = lax.top_k(kept640, _MAXT)
    selc = [jnp.take_along_axis(c, sidx2, axis=2) for c in comps]
    flat_scores = sel_scores.reshape(_B, _C * _MAXT)
    fs, fidx = lax.top_k(flat_scores, _MAXT)
    fcomp = [jnp.take_along_axis(c.reshape(_B, -1), fidx, axis=1) for c in selc]
    fcls = jnp.take_along_axis(
        jnp.broadcast_to(jnp.arange(_C, dtype=jnp.int32)[None, :, None], (_B, _C, _MAXT)).reshape(_B, -1),
        fidx,
        axis=1,
    )
    ok = fs > 0.0
    final_scores = jnp.where(ok, fs, 0.0)
    final_boxes = jnp.where(ok[..., None], jnp.stack(fcomp, axis=-1), 0.0)
    final_labels = jnp.where(ok, fcls, 0).astype(jnp.float32)
    return final_boxes, final_labels, final_scores


# speedup vs baseline: 12.9543x; 12.9543x over previous
import functools

import jax
import jax.numpy as jnp
from jax import lax
from jax.experimental import pallas as pl
from jax.experimental.pallas import tpu as pltpu

_N = 20000
_NPAD = 20480
_NBLK = 10240
_C = 21
_CPAD = 24
_B = 8
_PRE = 600
_PREPAD = 640
_INST = _B * _C
_LANES = 256
_MAXT = 200
_SCORE_TH = 0.5
_IOU_TH = 0.5


def _decode_body(deltas_ref, priors_ref, probs_ref, boxes_ref, scores_ref):
    d = deltas_ref[0]
    p = priors_ref[...]
    pw = p[3:4] - p[1:2]
    ph = p[2:3] - p[0:1]
    pcx = p[1:2] + 0.5 * pw
    pcy = p[0:1] + 0.5 * ph
    bw = jnp.exp(d[3:4] * 0.2) * pw
    bh = jnp.exp(d[2:3] * 0.2) * ph
    bcx = (d[1:2] * 0.1) * pw + pcx
    bcy = (d[0:1] * 0.1) * ph + pcy
    y1 = bcy - 0.5 * bh
    x1 = bcx - 0.5 * bw
    y2 = bh + y1
    x2 = bw + x1
    boxes_ref[0] = jnp.clip(jnp.concatenate([y1, x1, y2, x2], axis=0), 0.0, 1.0)
    pr = probs_ref[0]
    mx = jnp.max(pr, axis=0, keepdims=True)
    nonbg = pr[0:1] < mx
    scores_ref[0] = jnp.where(nonbg, pr, 0.0)


def _decode(pred_deltas, pred_label_probs, prior_boxes):
    deltas_t = jnp.pad(jnp.swapaxes(pred_deltas, 1, 2), ((0, 0), (0, 0), (0, _NPAD - _N)))
    priors_t = jnp.pad(prior_boxes.T, ((0, 0), (0, _NPAD - _N)))
    probs_t = jnp.pad(
        jnp.swapaxes(pred_label_probs, 1, 2),
        ((0, 0), (0, _CPAD - _C), (0, _NPAD - _N)),
        constant_values=-1.0,
    )
    grid = (_B, _NPAD // _NBLK)
    boxes_t, scores_t = pl.pallas_call(
        _decode_body,
        grid=grid,
        in_specs=[
            pl.BlockSpec((1, 4, _NBLK), lambda b, n: (b, 0, n)),
            pl.BlockSpec((4, _NBLK), lambda b, n: (0, n)),
            pl.BlockSpec((1, _CPAD, _NBLK), lambda b, n: (b, 0, n)),
        ],
        out_specs=[
            pl.BlockSpec((1, 4, _NBLK), lambda b, n: (b, 0, n)),
            pl.BlockSpec((1, _CPAD, _NBLK), lambda b, n: (b, 0, n)),
        ],
        out_shape=[
            jax.ShapeDtypeStruct((_B, 4, _NPAD), jnp.float32),
            jax.ShapeDtypeStruct((_B, _CPAD, _NPAD), jnp.float32),
        ],
    )(deltas_t, priors_t, probs_t)
    return boxes_t, scores_t


def _nms_body(y1_ref, x1_ref, y2_ref, x2_ref, sc_ref, out_ref, area_ref, keep_ref):
    y1 = y1_ref[...]
    x1 = x1_ref[...]
    y2 = y2_ref[...]
    x2 = x2_ref[...]
    area_ref[...] = (y2 - y1) * (x2 - x1)
    keep_ref[...] = jnp.ones_like(y1)

    _BK = 64
    for b0 in range(0, _PRE, _BK):
        nsteps = min(_BK, _PRE - b0)
        suf = _PREPAD - b0
        sy1 = y1_ref[b0:, :]
        sx1 = x1_ref[b0:, :]
        sy2 = y2_ref[b0:, :]
        sx2 = x2_ref[b0:, :]
        sarea = area_ref[b0:, :]
        rows = lax.broadcasted_iota(jnp.int32, (suf, _LANES), 0) + b0

        def body(i, _, sy1=sy1, sx1=sx1, sy2=sy2, sx2=sx2, sarea=sarea, rows=rows, b0=b0, suf=suf):
            ry1 = y1_ref[pl.ds(i, 1), :]
            rx1 = x1_ref[pl.ds(i, 1), :]
            ry2 = y2_ref[pl.ds(i, 1), :]
            rx2 = x2_ref[pl.ds(i, 1), :]
            rsc = sc_ref[pl.ds(i, 1), :]
            rkeep = keep_ref[pl.ds(i, 1), :]
            rarea = area_ref[pl.ds(i, 1), :]
            can = (rkeep > 0.0) & (rsc > _SCORE_TH)
            ih = jnp.maximum(jnp.minimum(sy2, ry2) - jnp.maximum(sy1, ry1), 0.0)
            iw = jnp.maximum(jnp.minimum(sx2, rx2) - jnp.maximum(sx1, rx1), 0.0)
            inter = ih * iw
            union = sarea + rarea - inter
            iou = inter / jnp.maximum(union, 1e-8)
            sup = (iou > _IOU_TH) & (rows != i) & can
            keep_ref[b0:, :] = jnp.where(sup, 0.0, keep_ref[b0:, :])
            return 0

        lax.fori_loop(b0, b0 + nsteps, body, 0)
    sc = sc_ref[...]
    out_ref[...] = jnp.where((keep_ref[...] > 0.0) & (sc > _SCORE_TH), sc, -1.0)


def _nms(y1l, x1l, y2l, x2l, scl):
    return pl.pallas_call(
        _nms_body,
        out_shape=jax.ShapeDtypeStruct((_PREPAD, _LANES), jnp.float32),
        scratch_shapes=[
            pltpu.VMEM((_PREPAD, _LANES), jnp.float32),
            pltpu.VMEM((_PREPAD, _LANES), jnp.float32),
        ],
    )(y1l, x1l, y2l, x2l, scl)


def _to_lane(a, pad_val):
    a = a.reshape(_INST, _PRE).T
    return jnp.pad(a, ((0, _PREPAD - _PRE), (0, _LANES - _INST)), constant_values=pad_val)


def kernel(pred_deltas, pred_label_probs, prior_boxes):
    boxes_t, scores_t = _decode(pred_deltas, pred_label_probs, prior_boxes)
    scores21 = scores_t[:, :_C, :]
    ts, idx = lax.top_k(scores21, _PRE)
    bsel = jnp.take_along_axis(boxes_t[:, :, None, :], idx[:, None, :, :], axis=3)
    comps = [bsel[:, k] for k in range(4)]
    y1l, x1l, y2l, x2l = [_to_lane(c, 0.0) for c in comps]
    scl = _to_lane(ts, -1.0)
    kept = _nms(y1l, x1l, y2l, x2l, scl)
    kept600 = kept[:_PRE, :_INST].T.reshape(_B, _C, _PRE)
    sel_scores, sidx = lax.top_k(kept600, _MAXT)
    selc = [jnp.take_along_axis(c, sidx, axis=2) for c in comps]
    flat_scores = sel_scores.reshape(_B, _C * _MAXT)
    fs, fidx = lax.top_k(flat_scores, _MAXT)
    fcomp = [jnp.take_along_axis(c.reshape(_B, -1), fidx, axis=1) for c in selc]
    fcls = jnp.take_along_axis(
        jnp.broadcast_to(jnp.arange(_C, dtype=jnp.int32)[None, :, None], (_B, _C, _MAXT)).reshape(_B, -1),
        fidx,
        axis=1,
    )
    ok = fs > 0.0
    final_scores = jnp.where(ok, fs, 0.0)
    final_boxes = jnp.where(ok[..., None], jnp.stack(fcomp, axis=-1), 0.0)
    final_labels = jnp.where(ok, fcls, 0).astype(jnp.float32)
    return final_boxes, final_labels, final_scores
